# baseline (speedup 1.0000x reference)
import functools

import jax
import jax.numpy as jnp
from jax import lax
from jax.experimental import pallas as pl
from jax.experimental.pallas import tpu as pltpu

TN = 512
TH = 512
LANES = 128


def _stage_a(x_ref, wg_ref, bg_ref, s_ref, gsum_ref, *, tn, e_num):
    n = pl.program_id(1)
    xb = x_ref[0]
    wg = wg_ref[...]
    logits = lax.dot_general(wg, xb, (((1,), (1,)), ((), ())),
                             preferred_element_type=jnp.float32)
    logits = logits + bg_ref[0][:, None]
    ids = lax.broadcasted_iota(jnp.int32, (e_num, tn), 0)
    m1 = jnp.max(logits, axis=0, keepdims=True)
    i1 = jnp.min(jnp.where(logits == m1, ids, e_num), axis=0, keepdims=True)
    mask1 = ids == i1
    masked = jnp.where(mask1, -jnp.inf, logits)
    m2 = jnp.max(masked, axis=0, keepdims=True)
    i2 = jnp.min(jnp.where(masked == m2, ids, e_num), axis=0, keepdims=True)
    mask2 = ids == i2
    g1 = 1.0 / (1.0 + jnp.exp(m2 - m1))
    gates = jnp.where(mask1, g1, 0.0) + jnp.where(mask2, 1.0 - g1, 0.0)
    sc = lax.dot_general(gates, xb, (((1,), (0,)), ((), ())),
                         preferred_element_type=jnp.float32)
    gs = jnp.sum(gates, axis=1)[None, None, :]

    @pl.when(n == 0)
    def _():
        s_ref[0] = sc
        gsum_ref[...] = gs

    @pl.when(n > 0)
    def _():
        s_ref[0] = s_ref[0] + sc
        gsum_ref[...] = gsum_ref[...] + gs


RG = 64


def _stage_b(wea_ref, web_ref, s0_ref, s1_ref, be_ref, gsum_ref, out_ref,
             acc_ref, accb_ref, *, th, e_num, d, b_num):
    e = pl.program_id(1)
    be_blk = be_ref[0, 0]
    gs = gsum_ref[:, 0, :]
    eids = lax.broadcasted_iota(jnp.int32, (b_num, e_num), 1)
    gse = jnp.sum(jnp.where(eids == e, gs, 0.0), axis=1)
    s0 = s0_ref[0]
    s1 = s1_ref[0]

    @pl.when(e == 0)
    def _():
        acc_ref[...] = jnp.zeros((b_num, th, LANES), jnp.float32)
        accb_ref[...] = jnp.zeros((b_num, th), jnp.float32)

    n_chunks = d // LANES
    half = th // 2
    for r in range(th // RG):
        rows = slice(r * RG, (r + 1) * RG)
        if (r + 1) * RG <= half:
            w_ref, wrows = wea_ref, rows
        else:
            w_ref = web_ref
            wrows = slice(r * RG - half, (r + 1) * RG - half)
        acc0 = acc_ref[0, rows]
        acc1 = acc_ref[1, rows]
        for k in range(n_chunks):
            cols = slice(k * LANES, (k + 1) * LANES)
            wv = w_ref[0, wrows, cols]
            acc0 = acc0 + wv * s0[:, cols]
            acc1 = acc1 + wv * s1[:, cols]
        acc_ref[0, rows] = acc0
        acc_ref[1, rows] = acc1
    accb_ref[...] = accb_ref[...] + gse[:, None] * be_blk[None, :]

    @pl.when(e == e_num - 1)
    def _():
        rows = [jnp.sum(acc_ref[b], axis=-1) + accb_ref[b]
                for b in range(b_num)]
        out_ref[...] = jnp.stack(rows, axis=0)


def kernel(x, Wg, bg, We, be):
    B, N, D = x.shape
    E, H, _ = We.shape
    tn = min(TN, N)
    th = min(TH, H)
    bg2 = bg.reshape(1, E)

    S, Gsum = pl.pallas_call(
        functools.partial(_stage_a, tn=tn, e_num=E),
        grid=(B, N // tn),
        in_specs=[
            pl.BlockSpec((1, tn, D), lambda b, n: (b, n, 0)),
            pl.BlockSpec((E, D), lambda b, n: (0, 0)),
            pl.BlockSpec((1, E), lambda b, n: (0, 0)),
        ],
        out_specs=[
            pl.BlockSpec((1, E, D), lambda b, n: (b, 0, 0)),
            pl.BlockSpec((1, 1, E), lambda b, n: (b, 0, 0)),
        ],
        out_shape=[
            jax.ShapeDtypeStruct((B, E, D), jnp.float32),
            jax.ShapeDtypeStruct((B, 1, E), jnp.float32),
        ],
        compiler_params=pltpu.CompilerParams(
            dimension_semantics=("parallel", "arbitrary")),
    )(x, Wg, bg2)

    S2 = S.reshape(B * E, 1, D)
    out = pl.pallas_call(
        functools.partial(_stage_b, th=th, e_num=E, d=D, b_num=B),
        grid=(H // th, E),
        in_specs=[
            pl.BlockSpec((1, th // 2, D), lambda h, e: (e, 2 * h, 0)),
            pl.BlockSpec((1, th // 2, D), lambda h, e: (e, 2 * h + 1, 0)),
            pl.BlockSpec((1, 1, D), lambda h, e: (e, 0, 0)),
            pl.BlockSpec((1, 1, D), lambda h, e: (e + E, 0, 0)),
            pl.BlockSpec((1, 1, th), lambda h, e: (e, 0, h)),
            pl.BlockSpec((B, 1, E), lambda h, e: (0, 0, 0)),
        ],
        out_specs=pl.BlockSpec((B, th), lambda h, e: (0, h)),
        out_shape=jax.ShapeDtypeStruct((B, H), jnp.float32),
        scratch_shapes=[
            pltpu.VMEM((B, th, LANES), jnp.float32),
            pltpu.VMEM((B, th), jnp.float32),
        ],
        compiler_params=pltpu.CompilerParams(
            dimension_semantics=("parallel", "arbitrary")),
    )(We, We, S2, S2, be.reshape(E, 1, H), Gsum)
    return out


# speedup vs baseline: 1.1489x; 1.1489x over previous
import functools

import jax
import jax.numpy as jnp
from jax import lax
from jax.experimental import pallas as pl
from jax.experimental.pallas import tpu as pltpu

TN = 512
TH = 1024
LANES = 128


def _stage_a(x_ref, wg_ref, bg_ref, s_ref, gsum_ref, *, tn, e_num):
    n = pl.program_id(1)
    xb = x_ref[0]
    wg = wg_ref[...]
    logits = lax.dot_general(wg, xb, (((1,), (1,)), ((), ())),
                             preferred_element_type=jnp.float32)
    logits = logits + bg_ref[0][:, None]
    ids = lax.broadcasted_iota(jnp.int32, (e_num, tn), 0)
    m1 = jnp.max(logits, axis=0, keepdims=True)
    i1 = jnp.min(jnp.where(logits == m1, ids, e_num), axis=0, keepdims=True)
    mask1 = ids == i1
    masked = jnp.where(mask1, -jnp.inf, logits)
    m2 = jnp.max(masked, axis=0, keepdims=True)
    i2 = jnp.min(jnp.where(masked == m2, ids, e_num), axis=0, keepdims=True)
    mask2 = ids == i2
    g1 = 1.0 / (1.0 + jnp.exp(m2 - m1))
    gates = jnp.where(mask1, g1, 0.0) + jnp.where(mask2, 1.0 - g1, 0.0)
    sc = lax.dot_general(gates, xb, (((1,), (0,)), ((), ())),
                         preferred_element_type=jnp.float32)
    gs = jnp.sum(gates, axis=1)[None, None, :]

    @pl.when(n == 0)
    def _():
        s_ref[0] = sc
        gsum_ref[...] = gs

    @pl.when(n > 0)
    def _():
        s_ref[0] = s_ref[0] + sc
        gsum_ref[...] = gsum_ref[...] + gs


RG = 64


def _stage_b(wea_ref, web_ref, s_ref, be_ref, gsum_ref, out_ref,
             acc_ref, accb_ref, *, th, e_num, d, b_num):
    e = pl.program_id(1)
    be_blk = be_ref[pl.ds(e, 1), 0]
    gs = gsum_ref[:, 0, :]
    eids = lax.broadcasted_iota(jnp.int32, (b_num, e_num), 1)
    gse = jnp.sum(jnp.where(eids == e, gs, 0.0), axis=1)
    s0 = s_ref[pl.ds(e, 1), 0]
    s1 = s_ref[pl.ds(e + e_num, 1), 0]

    @pl.when(e == 0)
    def _():
        acc_ref[...] = jnp.zeros((b_num, th, LANES), jnp.float32)
        accb_ref[...] = jnp.zeros((b_num, th), jnp.float32)

    n_chunks = d // LANES
    half = th // 2
    for r in range(th // RG):
        rows = slice(r * RG, (r + 1) * RG)
        if (r + 1) * RG <= half:
            w_ref, wrows = wea_ref, rows
        else:
            w_ref = web_ref
            wrows = slice(r * RG - half, (r + 1) * RG - half)
        acc0 = acc_ref[0, rows]
        acc1 = acc_ref[1, rows]
        for k in range(n_chunks):
            cols = slice(k * LANES, (k + 1) * LANES)
            wv = w_ref[0, wrows, cols]
            acc0 = acc0 + wv * s0[:, cols]
            acc1 = acc1 + wv * s1[:, cols]
        acc_ref[0, rows] = acc0
        acc_ref[1, rows] = acc1
    accb_ref[...] = accb_ref[...] + gse[:, None] * be_blk
    @pl.when(e == e_num - 1)
    def _():
        rows = [jnp.sum(acc_ref[b], axis=-1) + accb_ref[b]
                for b in range(b_num)]
        out_ref[...] = jnp.stack(rows, axis=0)


def kernel(x, Wg, bg, We, be):
    B, N, D = x.shape
    E, H, _ = We.shape
    tn = min(TN, N)
    th = min(TH, H)
    bg2 = bg.reshape(1, E)

    S, Gsum = pl.pallas_call(
        functools.partial(_stage_a, tn=tn, e_num=E),
        grid=(B, N // tn),
        in_specs=[
            pl.BlockSpec((1, tn, D), lambda b, n: (b, n, 0)),
            pl.BlockSpec((E, D), lambda b, n: (0, 0)),
            pl.BlockSpec((1, E), lambda b, n: (0, 0)),
        ],
        out_specs=[
            pl.BlockSpec((1, E, D), lambda b, n: (b, 0, 0)),
            pl.BlockSpec((1, 1, E), lambda b, n: (b, 0, 0)),
        ],
        out_shape=[
            jax.ShapeDtypeStruct((B, E, D), jnp.float32),
            jax.ShapeDtypeStruct((B, 1, E), jnp.float32),
        ],
        compiler_params=pltpu.CompilerParams(
            dimension_semantics=("parallel", "arbitrary")),
    )(x, Wg, bg2)

    S2 = S.reshape(B * E, 1, D)
    out = pl.pallas_call(
        functools.partial(_stage_b, th=th, e_num=E, d=D, b_num=B),
        grid=(H // th, E),
        in_specs=[
            pl.BlockSpec((1, th // 2, D), lambda h, e: (e, 2 * h, 0)),
            pl.BlockSpec((1, th // 2, D), lambda h, e: (e, 2 * h + 1, 0)),
            pl.BlockSpec((B * E, 1, D), lambda h, e: (0, 0, 0)),
            pl.BlockSpec((E, 1, th), lambda h, e: (0, 0, h)),
            pl.BlockSpec((B, 1, E), lambda h, e: (0, 0, 0)),
        ],
        out_specs=pl.BlockSpec((B, th), lambda h, e: (0, h)),
        out_shape=jax.ShapeDtypeStruct((B, H), jnp.float32),
        scratch_shapes=[
            pltpu.VMEM((B, th, LANES), jnp.float32),
            pltpu.VMEM((B, th), jnp.float32),
        ],
        compiler_params=pltpu.CompilerParams(
            dimension_semantics=("parallel", "arbitrary")),
    )(We, We, S2, be.reshape(E, 1, H), Gsum)
    return out


# speedup vs baseline: 1.1515x; 1.0022x over previous
import functools

import jax
import jax.numpy as jnp
from jax import lax
from jax.experimental import pallas as pl
from jax.experimental.pallas import tpu as pltpu

TN = 1024
TH = 2048
LANES = 128


def _stage_a(x_ref, wg_ref, bg_ref, s_ref, gsum_ref, *, tn, e_num):
    n = pl.program_id(1)
    xb = x_ref[0]
    wg = wg_ref[...]
    logits = lax.dot_general(wg, xb, (((1,), (1,)), ((), ())),
                             preferred_element_type=jnp.float32)
    logits = logits + bg_ref[0][:, None]
    ids = lax.broadcasted_iota(jnp.int32, (e_num, tn), 0)
    m1 = jnp.max(logits, axis=0, keepdims=True)
    i1 = jnp.min(jnp.where(logits == m1, ids, e_num), axis=0, keepdims=True)
    mask1 = ids == i1
    masked = jnp.where(mask1, -jnp.inf, logits)
    m2 = jnp.max(masked, axis=0, keepdims=True)
    i2 = jnp.min(jnp.where(masked == m2, ids, e_num), axis=0, keepdims=True)
    mask2 = ids == i2
    g1 = 1.0 / (1.0 + jnp.exp(m2 - m1))
    gates = jnp.where(mask1, g1, 0.0) + jnp.where(mask2, 1.0 - g1, 0.0)
    sc = lax.dot_general(gates, xb, (((1,), (0,)), ((), ())),
                         preferred_element_type=jnp.float32)
    gs = jnp.sum(gates, axis=1)[None, None, :]

    @pl.when(n == 0)
    def _():
        s_ref[0] = sc
        gsum_ref[...] = gs

    @pl.when(n > 0)
    def _():
        s_ref[0] = s_ref[0] + sc
        gsum_ref[...] = gsum_ref[...] + gs


RG = 64


def _stage_b(wea_ref, web_ref, s_ref, be_ref, gsum_ref, out_ref,
             acc_ref, accb_ref, *, th, e_num, d, b_num):
    e = pl.program_id(1)
    be_blk = be_ref[pl.ds(e, 1), 0]
    gs = gsum_ref[:, 0, :]
    eids = lax.broadcasted_iota(jnp.int32, (b_num, e_num), 1)
    gse = jnp.sum(jnp.where(eids == e, gs, 0.0), axis=1)
    s0 = s_ref[pl.ds(e, 1), 0]
    s1 = s_ref[pl.ds(e + e_num, 1), 0]

    @pl.when(e == 0)
    def _():
        acc_ref[...] = jnp.zeros((b_num, th, LANES), jnp.float32)
        accb_ref[...] = jnp.zeros((b_num, th), jnp.float32)

    n_chunks = d // LANES
    half = th // 2
    for r in range(th // RG):
        rows = slice(r * RG, (r + 1) * RG)
        if (r + 1) * RG <= half:
            w_ref, wrows = wea_ref, rows
        else:
            w_ref = web_ref
            wrows = slice(r * RG - half, (r + 1) * RG - half)
        acc0 = acc_ref[0, rows]
        acc1 = acc_ref[1, rows]
        for k in range(n_chunks):
            cols = slice(k * LANES, (k + 1) * LANES)
            wv = w_ref[0, wrows, cols]
            acc0 = acc0 + wv * s0[:, cols]
            acc1 = acc1 + wv * s1[:, cols]
        acc_ref[0, rows] = acc0
        acc_ref[1, rows] = acc1
    accb_ref[...] = accb_ref[...] + gse[:, None] * be_blk
    @pl.when(e == e_num - 1)
    def _():
        rows = [jnp.sum(acc_ref[b], axis=-1) + accb_ref[b]
                for b in range(b_num)]
        out_ref[...] = jnp.stack(rows, axis=0)


def kernel(x, Wg, bg, We, be):
    B, N, D = x.shape
    E, H, _ = We.shape
    tn = min(TN, N)
    th = min(TH, H)
    bg2 = bg.reshape(1, E)

    S, Gsum = pl.pallas_call(
        functools.partial(_stage_a, tn=tn, e_num=E),
        grid=(B, N // tn),
        in_specs=[
            pl.BlockSpec((1, tn, D), lambda b, n: (b, n, 0)),
            pl.BlockSpec((E, D), lambda b, n: (0, 0)),
            pl.BlockSpec((1, E), lambda b, n: (0, 0)),
        ],
        out_specs=[
            pl.BlockSpec((1, E, D), lambda b, n: (b, 0, 0)),
            pl.BlockSpec((1, 1, E), lambda b, n: (b, 0, 0)),
        ],
        out_shape=[
            jax.ShapeDtypeStruct((B, E, D), jnp.float32),
            jax.ShapeDtypeStruct((B, 1, E), jnp.float32),
        ],
        compiler_params=pltpu.CompilerParams(
            dimension_semantics=("parallel", "arbitrary")),
    )(x, Wg, bg2)

    S2 = S.reshape(B * E, 1, D)
    out = pl.pallas_call(
        functools.partial(_stage_b, th=th, e_num=E, d=D, b_num=B),
        grid=(H // th, E),
        in_specs=[
            pl.BlockSpec((1, th // 2, D), lambda h, e: (e, 2 * h, 0)),
            pl.BlockSpec((1, th // 2, D), lambda h, e: (e, 2 * h + 1, 0)),
            pl.BlockSpec((B * E, 1, D), lambda h, e: (0, 0, 0)),
            pl.BlockSpec((E, 1, th), lambda h, e: (0, 0, h)),
            pl.BlockSpec((B, 1, E), lambda h, e: (0, 0, 0)),
        ],
        out_specs=pl.BlockSpec((B, th), lambda h, e: (0, h)),
        out_shape=jax.ShapeDtypeStruct((B, H), jnp.float32),
        scratch_shapes=[
            pltpu.VMEM((B, th, LANES), jnp.float32),
            pltpu.VMEM((B, th), jnp.float32),
        ],
        compiler_params=pltpu.CompilerParams(
            dimension_semantics=("parallel", "arbitrary")),
    )(We, We, S2, be.reshape(E, 1, H), Gsum)
    return out
